# baseline (speedup 1.0000x reference)
import functools

import jax
import jax.numpy as jnp
from jax import lax
from jax.experimental import pallas as pl
from jax.experimental.pallas import tpu as pltpu
from jax.experimental.pallas import tpu_sc as plsc

_ROWS_PER_BLOCK = 1024
_CHUNK = 80


def _argmax_half(xb3):
    xb = xb3.reshape(xb3.shape[0], xb3.shape[2])
    vocab = xb.shape[-1]
    m = jnp.max(xb, axis=-1, keepdims=True)
    col = jax.lax.broadcasted_iota(jnp.int32, xb.shape, 1)
    return jnp.min(jnp.where(xb == m, col, vocab), axis=-1)


def _argmax_body(xa_ref, xb_ref, ta_ref, tb_ref):
    ta_ref[...] = _argmax_half(xa_ref[...])
    tb_ref[...] = _argmax_half(xb_ref[...])


def _compute_tokens(x3):
    rows, _, vocab = x3.shape
    grid = rows // (2 * _ROWS_PER_BLOCK)
    blk = (_ROWS_PER_BLOCK, 1, vocab)
    return pl.pallas_call(
        _argmax_body,
        grid=(grid,),
        in_specs=[
            pl.BlockSpec(blk, lambda i: (i, 0, 0)),
            pl.BlockSpec(blk, lambda i, g=grid: (i + g, 0, 0)),
        ],
        out_specs=[
            pl.BlockSpec((_ROWS_PER_BLOCK,), lambda i: (i,)),
            pl.BlockSpec((_ROWS_PER_BLOCK,), lambda i: (i,)),
        ],
        out_shape=[
            jax.ShapeDtypeStruct((rows // 2,), jnp.int32),
            jax.ShapeDtypeStruct((rows // 2,), jnp.int32),
        ],
    )(x3, x3)


def _make_gather(rows, emb, n_workers, n_chunks):
    bpw = rows // n_workers
    half_workers = n_workers // 2

    def _gather_body(ta_hbm, tb_hbm, table_hbm, out_hbm, idx_v, rows_v, sem):
        wid = lax.axis_index("s") * 2 + lax.axis_index("c")
        @pl.when(wid < half_workers)
        def _():
            pltpu.sync_copy(ta_hbm.at[pl.ds(wid * bpw, bpw)], idx_v)

        @pl.when(wid >= half_workers)
        def _():
            pltpu.sync_copy(
                tb_hbm.at[pl.ds((wid - half_workers) * bpw, bpw)], idx_v)

        copies = [
            pltpu.async_copy(
                table_hbm.at[idx_v.at[pl.ds(j * _CHUNK, _CHUNK)]],
                rows_v.at[pl.ds(j * _CHUNK, _CHUNK)],
                sem,
            )
            for j in range(n_chunks)
        ]
        for cp in copies:
            cp.wait()
        pltpu.sync_copy(rows_v, out_hbm.at[pl.ds(wid * bpw, bpw)])

    mesh = plsc.VectorSubcoreMesh(core_axis_name="c", subcore_axis_name="s")
    return pl.kernel(
        _gather_body,
        mesh=mesh,
        compiler_params=pltpu.CompilerParams(use_tc_tiling_on_sc=False),
        out_type=jax.ShapeDtypeStruct((rows, emb), jnp.float32),
        scratch_types=[
            pltpu.VMEM((bpw,), jnp.int32),
            pltpu.VMEM((bpw, emb), jnp.float32),
            pltpu.SemaphoreType.DMA,
        ],
    )


def kernel(x, dictionary):
    b, n, vocab = x.shape
    emb = dictionary.shape[1]
    rows = b * n
    n_workers = 32
    n_chunks = rows // (n_workers * _CHUNK)

    ta, tb = _compute_tokens(x.reshape(rows, 1, vocab))
    out = _make_gather(rows, emb, n_workers, n_chunks)(ta, tb, dictionary)
    return out.reshape(b, n, emb)


# speedup vs baseline: 3.0332x; 3.0332x over previous
import functools

import jax
import jax.numpy as jnp
from jax import lax
from jax.experimental import pallas as pl
from jax.experimental.pallas import tpu as pltpu
from jax.experimental.pallas import tpu_sc as plsc

_BATCH_PER_BLOCK = 32
_CHUNK = 80


def _argmax_half(xb):
    vocab = xb.shape[-1]
    m = jnp.max(xb, axis=-1, keepdims=True)
    col = jax.lax.broadcasted_iota(jnp.int32, xb.shape, 2)
    return jnp.min(jnp.where(xb == m, col, vocab), axis=-1)


def _argmax_body(xa_ref, xb_ref, ta_ref, tb_ref):
    ta_ref[...] = _argmax_half(xa_ref[...])
    tb_ref[...] = _argmax_half(xb_ref[...])


def _compute_tokens(x):
    b, n, vocab = x.shape
    grid = b // (2 * _BATCH_PER_BLOCK)
    half = b // (2 * _BATCH_PER_BLOCK)
    blk = (_BATCH_PER_BLOCK, n, vocab)
    ta, tb = pl.pallas_call(
        _argmax_body,
        grid=(grid,),
        in_specs=[
            pl.BlockSpec(blk, lambda i: (i, 0, 0)),
            pl.BlockSpec(blk, lambda i, h=half: (i + h, 0, 0)),
        ],
        out_specs=[
            pl.BlockSpec((_BATCH_PER_BLOCK, n), lambda i: (i, 0)),
            pl.BlockSpec((_BATCH_PER_BLOCK, n), lambda i: (i, 0)),
        ],
        out_shape=[
            jax.ShapeDtypeStruct((b // 2, n), jnp.int32),
            jax.ShapeDtypeStruct((b // 2, n), jnp.int32),
        ],
    )(x, x)
    return ta, tb


def _make_gather(rows, emb, n_workers, n_chunks):
    bpw = rows // n_workers

    def _gather_body(tok_hbm, table_hbm, out_hbm, idx_v, rows_v, sem):
        wid = lax.axis_index("s") * 2 + lax.axis_index("c")
        pltpu.sync_copy(tok_hbm.at[pl.ds(wid * bpw, bpw)], idx_v)
        copies = [
            pltpu.async_copy(
                table_hbm.at[idx_v.at[pl.ds(j * _CHUNK, _CHUNK)]],
                rows_v.at[pl.ds(j * _CHUNK, _CHUNK)],
                sem,
            )
            for j in range(n_chunks)
        ]
        for cp in copies:
            cp.wait()
        pltpu.sync_copy(rows_v, out_hbm.at[pl.ds(wid * bpw, bpw)])

    mesh = plsc.VectorSubcoreMesh(core_axis_name="c", subcore_axis_name="s")
    return pl.kernel(
        _gather_body,
        mesh=mesh,
        compiler_params=pltpu.CompilerParams(use_tc_tiling_on_sc=False),
        out_type=jax.ShapeDtypeStruct((rows, emb), jnp.float32),
        scratch_types=[
            pltpu.VMEM((bpw,), jnp.int32),
            pltpu.VMEM((bpw, emb), jnp.float32),
            pltpu.SemaphoreType.DMA,
        ],
    )


def kernel(x, dictionary):
    b, n, vocab = x.shape
    emb = dictionary.shape[1]
    rows = b * n
    n_workers = 32
    n_chunks = rows // (n_workers * _CHUNK)

    ta, tb = _compute_tokens(x)
    tokens = jnp.concatenate([ta.reshape(rows // 2), tb.reshape(rows // 2)])
    out = _make_gather(rows, emb, n_workers, n_chunks)(tokens, dictionary)
    return out.reshape(b, n, emb)
